# baseline (speedup 1.0000x reference)
import functools

import numpy as np
import jax
import jax.numpy as jnp
from jax import lax
from jax.experimental import pallas as pl
from jax.experimental.pallas import tpu as pltpu
from jax.experimental.pallas import tpu_sc as plsc

N_LEVELS = 16
F_PER_LEVEL = 2
LOG2_T = 19
T = 1 << LOG2_T
BASE_RES = 16
PER_LEVEL_SCALE = 1.5
N_NEURONS = 64
IN_DIM = 2 + N_LEVELS * F_PER_LEVEL

RES = [int(np.floor(BASE_RES * (PER_LEVEL_SCALE ** l))) for l in range(N_LEVELS)]
HASHED = [(r + 1) * (r + 1) > T for r in RES]
PRIME1 = int(np.uint32(2654435761).view(np.int32))

NC, NS, LANES = 2, 16, 16
NW = NC * NS
NJ = N_LEVELS * 4
CHUNK = 128

N_STAGED = 7
STAGED_ROWS = [(RES[l] + 1) * (RES[l] + 1) for l in range(N_STAGED)]
SBASE = np.cumsum([0] + STAGED_ROWS).tolist()
N_ST = SBASE[-1]
NJS = N_STAGED * 4
NJH = NJ - NJS


def _phase_a(xv_v, yv_v, idx_v, w_v):
    def grp_a(g, c2):
        s = 16 * g
        xv = xv_v[pl.ds(s, LANES)]
        yv = yv_v[pl.ds(s, LANES)]
        for l in range(N_LEVELS):
            res = RES[l]
            px = xv * float(res)
            py = yv * float(res)
            ix = px.astype(jnp.int32)
            iy = py.astype(jnp.int32)
            fx = px - ix.astype(jnp.float32)
            fy = py - iy.astype(jnp.float32)
            wxs = (1.0 - fx, fx)
            wys = (1.0 - fy, fy)
            off = SBASE[l] if l < N_STAGED else l * T
            if HASHED[l]:
                hy0 = iy * PRIME1
                hy1 = (iy + 1) * PRIME1
                i00 = ((ix ^ hy0) & (T - 1)) + off
                i01 = ((ix ^ hy1) & (T - 1)) + off
                ix1 = ix + 1
                i10 = ((ix1 ^ hy0) & (T - 1)) + off
                i11 = ((ix1 ^ hy1) & (T - 1)) + off
            else:
                stride = res + 1
                i00 = ix + iy * stride + off
                i10 = i00 + 1
                i01 = i00 + stride
                i11 = i01 + 1
            corners = ((i00, 0, 0), (i01, 0, 1), (i10, 1, 0), (i11, 1, 1))
            for c, (idx, cx, cy) in enumerate(corners):
                j = 4 * l + c
                idx_v[pl.ds(j * CHUNK + s, LANES)] = idx
                w_v[pl.ds(j * CHUNK + s, LANES)] = wxs[cx] * wys[cy]
        return c2

    lax.fori_loop(0, CHUNK // LANES, grp_a, 0)


def _phase_c(cbase, xv_v, yv_v, idx_v, w_v, rpk_v, st_v, acc_v, enc_hbm):
    himask = jnp.int32(-65536)

    def grp_c(g, c2):
        s = 16 * g
        acc_v[0, pl.ds(s, LANES)] = xv_v[pl.ds(s, LANES)]
        acc_v[1, pl.ds(s, LANES)] = yv_v[pl.ds(s, LANES)]
        for l in range(N_LEVELS):
            a0 = jnp.zeros((LANES,), jnp.float32)
            a1 = jnp.zeros((LANES,), jnp.float32)
            for c in range(4):
                j = 4 * l + c
                wv = w_v[pl.ds(j * CHUNK + s, LANES)]
                if l < N_STAGED:
                    iv = idx_v[pl.ds(j * CHUNK + s, LANES)]
                    v = plsc.load_gather(st_v, [iv])
                else:
                    v = rpk_v[pl.ds((j - NJS) * CHUNK + s, LANES)]
                f0 = plsc.bitcast(v << 16, jnp.float32)
                f1 = plsc.bitcast(v & himask, jnp.float32)
                a0 = a0 + wv * f0
                a1 = a1 + wv * f1
            acc_v[2 + 2 * l, pl.ds(s, LANES)] = a0
            acc_v[3 + 2 * l, pl.ds(s, LANES)] = a1
        return c2

    lax.fori_loop(0, CHUNK // LANES, grp_c, 0)
    pltpu.sync_copy(acc_v, enc_hbm.at[:, pl.ds(cbase, CHUNK)])


def _enc_body(nchunk, xs_hbm, ys_hbm, tpk_hbm, st_hbm, enc_hbm,
              xv0, yv0, idx0, w0_, rpk0, xv1, yv1, idx1, w1_, rpk1,
              st_v, acc_v, sem0, sem1):
    pts_per_w = nchunk * CHUNK
    wid = lax.axis_index("s") * NC + lax.axis_index("c")
    base = wid * pts_per_w

    pltpu.sync_copy(st_hbm, st_v)

    def load_x(ci, xv_v, yv_v):
        cbase = base + ci * CHUNK
        pltpu.sync_copy(xs_hbm.at[pl.ds(cbase, CHUNK)], xv_v)
        pltpu.sync_copy(ys_hbm.at[pl.ds(cbase, CHUNK)], yv_v)

    def fire(idx_v, rpk_v, sem):
        pltpu.async_copy(tpk_hbm.at[idx_v.at[pl.ds(NJS * CHUNK, NJH * CHUNK)]],
                         rpk_v, sem)

    def drain(rpk_v, sem):
        pltpu.make_async_copy(tpk_hbm.at[pl.ds(0, NJH * CHUNK)], rpk_v, sem).wait()

    load_x(0, xv0, yv0)
    _phase_a(xv0, yv0, idx0, w0_)
    fire(idx0, rpk0, sem0)

    def body(ci2, carry):
        ci = 2 * ci2
        load_x(ci + 1, xv1, yv1)
        _phase_a(xv1, yv1, idx1, w1_)
        fire(idx1, rpk1, sem1)
        drain(rpk0, sem0)
        _phase_c(base + ci * CHUNK, xv0, yv0, idx0, w0_, rpk0, st_v, acc_v, enc_hbm)

        @pl.when(ci2 + 1 < nchunk // 2)
        def _():
            load_x(ci + 2, xv0, yv0)
            _phase_a(xv0, yv0, idx0, w0_)
            fire(idx0, rpk0, sem0)

        drain(rpk1, sem1)
        _phase_c(base + (ci + 1) * CHUNK, xv1, yv1, idx1, w1_, rpk1, st_v, acc_v,
                 enc_hbm)
        return carry

    lax.fori_loop(0, nchunk // 2, body, 0)


def _encode(xs, ys, tpk, st, n):
    nchunk = n // (NW * CHUNK)
    assert nchunk % 2 == 0
    mesh = plsc.VectorSubcoreMesh(core_axis_name="c", subcore_axis_name="s")
    return pl.kernel(
        functools.partial(_enc_body, nchunk),
        out_type=jax.ShapeDtypeStruct((IN_DIM, n), jnp.float32),
        mesh=mesh,
        compiler_params=pltpu.CompilerParams(needs_layout_passes=False),
        scratch_types=[
            pltpu.VMEM((CHUNK,), jnp.float32), pltpu.VMEM((CHUNK,), jnp.float32),
            pltpu.VMEM((NJ * CHUNK,), jnp.int32),
            pltpu.VMEM((NJ * CHUNK,), jnp.float32),
            pltpu.VMEM((NJH * CHUNK,), jnp.int32),
            pltpu.VMEM((CHUNK,), jnp.float32), pltpu.VMEM((CHUNK,), jnp.float32),
            pltpu.VMEM((NJ * CHUNK,), jnp.int32),
            pltpu.VMEM((NJ * CHUNK,), jnp.float32),
            pltpu.VMEM((NJH * CHUNK,), jnp.int32),
            pltpu.VMEM((N_ST,), jnp.int32),
            pltpu.VMEM((IN_DIM, CHUNK), jnp.float32),
            pltpu.SemaphoreType.DMA,
            pltpu.SemaphoreType.DMA,
        ],
    )(xs, ys, tpk, st)


def _mlp_body(enc_ref, w0_ref, b0_ref, w1_ref, b1_ref, w2_ref, b2_ref, out_ref):
    prec = lax.Precision.HIGHEST
    encT = enc_ref[...]
    h = lax.dot_general(encT, w0_ref[...], (((0,), (0,)), ((), ())),
                        preferred_element_type=jnp.float32, precision=prec)
    h = jnp.maximum(h + b0_ref[...], 0.0)
    h = lax.dot_general(h, w1_ref[...], (((1,), (0,)), ((), ())),
                        preferred_element_type=jnp.float32, precision=prec)
    h = jnp.maximum(h + b1_ref[...], 0.0)
    out = lax.dot_general(h, w2_ref[...], (((1,), (0,)), ((), ())),
                          preferred_element_type=jnp.float32, precision=prec)
    out_ref[...] = out + b2_ref[...]


def _mlp(enc, W0, b0, W1, b1, W2, b2, n):
    bn = 2048
    return pl.pallas_call(
        _mlp_body,
        grid=(n // bn,),
        in_specs=[
            pl.BlockSpec((IN_DIM, bn), lambda i: (0, i)),
            pl.BlockSpec((IN_DIM, N_NEURONS), lambda i: (0, 0)),
            pl.BlockSpec((1, N_NEURONS), lambda i: (0, 0)),
            pl.BlockSpec((N_NEURONS, N_NEURONS), lambda i: (0, 0)),
            pl.BlockSpec((1, N_NEURONS), lambda i: (0, 0)),
            pl.BlockSpec((N_NEURONS, 3), lambda i: (0, 0)),
            pl.BlockSpec((1, 3), lambda i: (0, 0)),
        ],
        out_specs=pl.BlockSpec((bn, 3), lambda i: (i, 0)),
        out_shape=jax.ShapeDtypeStruct((n, 3), jnp.float32),
    )(enc, W0, b0, W1, b1, W2, b2)


def kernel(x, table, W0, b0, W1, b1, W2, b2):
    n = x.shape[0]
    xs = x[:, 0]
    ys = x[:, 1]
    tpk = lax.bitcast_convert_type(
        table.astype(jnp.bfloat16).reshape(N_LEVELS * T, F_PER_LEVEL), jnp.int32)
    st = jnp.concatenate(
        [tpk[l * T:l * T + STAGED_ROWS[l]] for l in range(N_STAGED)])
    enc = _encode(xs, ys, tpk, st, n)
    return _mlp(enc, W0, b0.reshape(1, -1), W1, b1.reshape(1, -1),
                W2, b2.reshape(1, -1), n)


# speedup vs baseline: 1.5111x; 1.5111x over previous
import functools

import numpy as np
import jax
import jax.numpy as jnp
from jax import lax
from jax.experimental import pallas as pl
from jax.experimental.pallas import tpu as pltpu
from jax.experimental.pallas import tpu_sc as plsc

N_LEVELS = 16
F_PER_LEVEL = 2
LOG2_T = 19
T = 1 << LOG2_T
BASE_RES = 16
PER_LEVEL_SCALE = 1.5
N_NEURONS = 64
IN_DIM = 2 + N_LEVELS * F_PER_LEVEL

RES = [int(np.floor(BASE_RES * (PER_LEVEL_SCALE ** l))) for l in range(N_LEVELS)]
HASHED = [(r + 1) * (r + 1) > T for r in RES]
PRIME1 = int(np.uint32(2654435761).view(np.int32))

NC, NS, LANES = 2, 16, 16
NW = NC * NS
NJ = N_LEVELS * 4
CHUNK = 128

N_STAGED = 7
STAGED_ROWS = [(RES[l] + 1) * (RES[l] + 1) for l in range(N_STAGED)]
SBASE = np.cumsum([0] + STAGED_ROWS).tolist()
N_ST = SBASE[-1]
NJS = N_STAGED * 4
NJH = NJ - NJS


def _grid_setup(xv, yv, l):
    res = RES[l]
    px = xv * float(res)
    py = yv * float(res)
    ix = px.astype(jnp.int32)
    iy = py.astype(jnp.int32)
    fx = px - ix.astype(jnp.float32)
    fy = py - iy.astype(jnp.float32)
    return ix, iy, (1.0 - fx, fx), (1.0 - fy, fy)


_HIMASK = -65536


def _unpack_fma(v, wv, a0, a1):
    f0 = plsc.bitcast(v << 16, jnp.float32)
    f1 = plsc.bitcast(v & jnp.int32(_HIMASK), jnp.float32)
    return a0 + wv * f0, a1 + wv * f1


def _phase_a(xv_v, yv_v, idx_v, w_v):
    def grp_a(g, c2):
        s = 16 * g
        xv = xv_v[pl.ds(s, LANES)]
        yv = yv_v[pl.ds(s, LANES)]
        for l in range(N_STAGED, N_LEVELS):
            ix, iy, wxs, wys = _grid_setup(xv, yv, l)
            off = l * T
            if HASHED[l]:
                hy0 = iy * PRIME1
                hy1 = (iy + 1) * PRIME1
                i00 = ((ix ^ hy0) & (T - 1)) + off
                i01 = ((ix ^ hy1) & (T - 1)) + off
                ix1 = ix + 1
                i10 = ((ix1 ^ hy0) & (T - 1)) + off
                i11 = ((ix1 ^ hy1) & (T - 1)) + off
            else:
                stride = RES[l] + 1
                i00 = ix + iy * stride + off
                i10 = i00 + 1
                i01 = i00 + stride
                i11 = i01 + 1
            corners = ((i00, 0, 0), (i01, 0, 1), (i10, 1, 0), (i11, 1, 1))
            for c, (idx, cx, cy) in enumerate(corners):
                j = 4 * (l - N_STAGED) + c
                idx_v[pl.ds(j * CHUNK + s, LANES)] = idx
                w_v[pl.ds(j * CHUNK + s, LANES)] = wxs[cx] * wys[cy]
        return c2

    lax.fori_loop(0, CHUNK // LANES, grp_a, 0)


def _phase_s(xv_v, yv_v, st_v, acc_v):
    def grp_s(g, c2):
        s = 16 * g
        xv = xv_v[pl.ds(s, LANES)]
        yv = yv_v[pl.ds(s, LANES)]
        acc_v[0, pl.ds(s, LANES)] = xv
        acc_v[1, pl.ds(s, LANES)] = yv
        for l in range(N_STAGED):
            ix, iy, wxs, wys = _grid_setup(xv, yv, l)
            stride = RES[l] + 1
            i00 = ix + iy * stride + SBASE[l]
            i10 = i00 + 1
            i01 = i00 + stride
            i11 = i01 + 1
            a0 = jnp.zeros((LANES,), jnp.float32)
            a1 = jnp.zeros((LANES,), jnp.float32)
            for idx, cx, cy in ((i00, 0, 0), (i01, 0, 1), (i10, 1, 0), (i11, 1, 1)):
                v = plsc.load_gather(st_v, [idx])
                a0, a1 = _unpack_fma(v, wxs[cx] * wys[cy], a0, a1)
            acc_v[2 + 2 * l, pl.ds(s, LANES)] = a0
            acc_v[3 + 2 * l, pl.ds(s, LANES)] = a1
        return c2

    lax.fori_loop(0, CHUNK // LANES, grp_s, 0)


def _phase_c(cbase, w_v, rpk_v, acc_v, enc_hbm):
    def grp_c(g, c2):
        s = 16 * g
        for l in range(N_STAGED, N_LEVELS):
            a0 = jnp.zeros((LANES,), jnp.float32)
            a1 = jnp.zeros((LANES,), jnp.float32)
            for c in range(4):
                j = 4 * (l - N_STAGED) + c
                wv = w_v[pl.ds(j * CHUNK + s, LANES)]
                v = rpk_v[pl.ds(j * CHUNK + s, LANES)]
                a0, a1 = _unpack_fma(v, wv, a0, a1)
            acc_v[2 + 2 * l, pl.ds(s, LANES)] = a0
            acc_v[3 + 2 * l, pl.ds(s, LANES)] = a1
        return c2

    lax.fori_loop(0, CHUNK // LANES, grp_c, 0)
    pltpu.sync_copy(acc_v, enc_hbm.at[:, pl.ds(cbase, CHUNK)])


def _enc_body(nchunk, xs_hbm, ys_hbm, tpk_hbm, st_hbm, enc_hbm,
              xv0, yv0, idx0, w0_, rpk0, xv1, yv1, idx1, w1_, rpk1,
              st_v, acc_v, sem0, sem1):
    pts_per_w = nchunk * CHUNK
    wid = lax.axis_index("s") * NC + lax.axis_index("c")
    base = wid * pts_per_w

    pltpu.sync_copy(st_hbm, st_v)

    def load_x(ci, xv_v, yv_v):
        cbase = base + ci * CHUNK
        pltpu.sync_copy(xs_hbm.at[pl.ds(cbase, CHUNK)], xv_v)
        pltpu.sync_copy(ys_hbm.at[pl.ds(cbase, CHUNK)], yv_v)

    def fire(idx_v, rpk_v, sem):
        pltpu.async_copy(tpk_hbm.at[idx_v], rpk_v, sem)

    def drain(rpk_v, sem):
        pltpu.make_async_copy(tpk_hbm.at[pl.ds(0, NJH * CHUNK)], rpk_v, sem).wait()

    load_x(0, xv0, yv0)
    _phase_a(xv0, yv0, idx0, w0_)
    fire(idx0, rpk0, sem0)

    def body(ci2, carry):
        ci = 2 * ci2
        load_x(ci + 1, xv1, yv1)
        _phase_a(xv1, yv1, idx1, w1_)
        fire(idx1, rpk1, sem1)
        _phase_s(xv0, yv0, st_v, acc_v)
        drain(rpk0, sem0)
        _phase_c(base + ci * CHUNK, w0_, rpk0, acc_v, enc_hbm)

        @pl.when(ci2 + 1 < nchunk // 2)
        def _():
            load_x(ci + 2, xv0, yv0)
            _phase_a(xv0, yv0, idx0, w0_)
            fire(idx0, rpk0, sem0)

        _phase_s(xv1, yv1, st_v, acc_v)
        drain(rpk1, sem1)
        _phase_c(base + (ci + 1) * CHUNK, w1_, rpk1, acc_v, enc_hbm)
        return carry

    lax.fori_loop(0, nchunk // 2, body, 0)


def _encode(xs, ys, tpk, st, n):
    nchunk = n // (NW * CHUNK)
    assert nchunk % 2 == 0
    mesh = plsc.VectorSubcoreMesh(core_axis_name="c", subcore_axis_name="s")
    return pl.kernel(
        functools.partial(_enc_body, nchunk),
        out_type=jax.ShapeDtypeStruct((IN_DIM, n), jnp.float32),
        mesh=mesh,
        compiler_params=pltpu.CompilerParams(needs_layout_passes=False),
        scratch_types=[
            pltpu.VMEM((CHUNK,), jnp.float32), pltpu.VMEM((CHUNK,), jnp.float32),
            pltpu.VMEM((NJH * CHUNK,), jnp.int32),
            pltpu.VMEM((NJH * CHUNK,), jnp.float32),
            pltpu.VMEM((NJH * CHUNK,), jnp.int32),
            pltpu.VMEM((CHUNK,), jnp.float32), pltpu.VMEM((CHUNK,), jnp.float32),
            pltpu.VMEM((NJH * CHUNK,), jnp.int32),
            pltpu.VMEM((NJH * CHUNK,), jnp.float32),
            pltpu.VMEM((NJH * CHUNK,), jnp.int32),
            pltpu.VMEM((N_ST,), jnp.int32),
            pltpu.VMEM((IN_DIM, CHUNK), jnp.float32),
            pltpu.SemaphoreType.DMA,
            pltpu.SemaphoreType.DMA,
        ],
    )(xs, ys, tpk, st)


def _mlp_body(enc_ref, w0_ref, b0_ref, w1_ref, b1_ref, w2_ref, b2_ref, out_ref):
    prec = lax.Precision.DEFAULT
    encT = enc_ref[...]
    h = lax.dot_general(encT, w0_ref[...], (((0,), (0,)), ((), ())),
                        preferred_element_type=jnp.float32, precision=prec)
    h = jnp.maximum(h + b0_ref[...], 0.0)
    h = lax.dot_general(h, w1_ref[...], (((1,), (0,)), ((), ())),
                        preferred_element_type=jnp.float32, precision=prec)
    h = jnp.maximum(h + b1_ref[...], 0.0)
    out = lax.dot_general(h, w2_ref[...], (((1,), (0,)), ((), ())),
                          preferred_element_type=jnp.float32, precision=prec)
    out_ref[...] = out + b2_ref[...]


def _mlp(enc, W0, b0, W1, b1, W2, b2, n):
    bn = 2048
    return pl.pallas_call(
        _mlp_body,
        grid=(n // bn,),
        in_specs=[
            pl.BlockSpec((IN_DIM, bn), lambda i: (0, i)),
            pl.BlockSpec((IN_DIM, N_NEURONS), lambda i: (0, 0)),
            pl.BlockSpec((1, N_NEURONS), lambda i: (0, 0)),
            pl.BlockSpec((N_NEURONS, N_NEURONS), lambda i: (0, 0)),
            pl.BlockSpec((1, N_NEURONS), lambda i: (0, 0)),
            pl.BlockSpec((N_NEURONS, 3), lambda i: (0, 0)),
            pl.BlockSpec((1, 3), lambda i: (0, 0)),
        ],
        out_specs=pl.BlockSpec((bn, 3), lambda i: (i, 0)),
        out_shape=jax.ShapeDtypeStruct((n, 3), jnp.float32),
    )(enc, W0, b0, W1, b1, W2, b2)


def kernel(x, table, W0, b0, W1, b1, W2, b2):
    n = x.shape[0]
    xs = x[:, 0]
    ys = x[:, 1]
    tpk = lax.bitcast_convert_type(
        table.astype(jnp.bfloat16).reshape(N_LEVELS * T, F_PER_LEVEL), jnp.int32)
    st = jnp.concatenate(
        [tpk[l * T:l * T + STAGED_ROWS[l]] for l in range(N_STAGED)])
    enc = _encode(xs, ys, tpk, st, n)
    return _mlp(enc, W0, b0.reshape(1, -1), W1, b1.reshape(1, -1),
                W2, b2.reshape(1, -1), n)


# speedup vs baseline: 1.6317x; 1.0798x over previous
import functools

import numpy as np
import jax
import jax.numpy as jnp
from jax import lax
from jax.experimental import pallas as pl
from jax.experimental.pallas import tpu as pltpu
from jax.experimental.pallas import tpu_sc as plsc

N_LEVELS = 16
F_PER_LEVEL = 2
LOG2_T = 19
T = 1 << LOG2_T
BASE_RES = 16
PER_LEVEL_SCALE = 1.5
N_NEURONS = 64
IN_DIM = 2 + N_LEVELS * F_PER_LEVEL

RES = [int(np.floor(BASE_RES * (PER_LEVEL_SCALE ** l))) for l in range(N_LEVELS)]
HASHED = [(r + 1) * (r + 1) > T for r in RES]
PRIME1 = int(np.uint32(2654435761).view(np.int32))

NC, NS, LANES = 2, 16, 16
NW = NC * NS
NJ = N_LEVELS * 4
CHUNK = 128

N_STAGED = 7
STAGED_ROWS = [(RES[l] + 1) * (RES[l] + 1) for l in range(N_STAGED)]
SBASE = np.cumsum([0] + STAGED_ROWS).tolist()
N_ST = SBASE[-1]
NJS = N_STAGED * 4
NJH = NJ - NJS


def _grid_setup(xv, yv, l):
    res = RES[l]
    px = xv * float(res)
    py = yv * float(res)
    ix = px.astype(jnp.int32)
    iy = py.astype(jnp.int32)
    fx = px - ix.astype(jnp.float32)
    fy = py - iy.astype(jnp.float32)
    return ix, iy, (1.0 - fx, fx), (1.0 - fy, fy)


_HIMASK = -65536


def _unpack_fma(v, wv, a0, a1):
    f0 = plsc.bitcast(v << 16, jnp.float32)
    f1 = plsc.bitcast(v & jnp.int32(_HIMASK), jnp.float32)
    return a0 + wv * f0, a1 + wv * f1


def _phase_a(xv_v, yv_v, idx_v, w_v):
    def grp_a(g, c2):
        s = 16 * g
        xv = xv_v[pl.ds(s, LANES)]
        yv = yv_v[pl.ds(s, LANES)]
        for l in range(N_STAGED, N_LEVELS):
            ix, iy, wxs, wys = _grid_setup(xv, yv, l)
            off = l * T
            if HASHED[l]:
                hy0 = iy * PRIME1
                hy1 = (iy + 1) * PRIME1
                i00 = ((ix ^ hy0) & (T - 1)) + off
                i01 = ((ix ^ hy1) & (T - 1)) + off
                ix1 = ix + 1
                i10 = ((ix1 ^ hy0) & (T - 1)) + off
                i11 = ((ix1 ^ hy1) & (T - 1)) + off
            else:
                stride = RES[l] + 1
                i00 = ix + iy * stride + off
                i10 = i00 + 1
                i01 = i00 + stride
                i11 = i01 + 1
            corners = ((i00, 0, 0), (i01, 0, 1), (i10, 1, 0), (i11, 1, 1))
            for c, (idx, cx, cy) in enumerate(corners):
                j = 4 * (l - N_STAGED) + c
                idx_v[pl.ds(j * CHUNK + s, LANES)] = idx
                w_v[pl.ds(j * CHUNK + s, LANES)] = wxs[cx] * wys[cy]
        return c2

    lax.fori_loop(0, CHUNK // LANES, grp_a, 0)


def _phase_s(xv_v, yv_v, st_v, acc_v):
    def grp_s(g, c2):
        s = 16 * g
        xv = xv_v[pl.ds(s, LANES)]
        yv = yv_v[pl.ds(s, LANES)]
        acc_v[0, pl.ds(s, LANES)] = xv
        acc_v[1, pl.ds(s, LANES)] = yv
        for l in range(N_STAGED):
            ix, iy, wxs, wys = _grid_setup(xv, yv, l)
            stride = RES[l] + 1
            i00 = ix + iy * stride + SBASE[l]
            i10 = i00 + 1
            i01 = i00 + stride
            i11 = i01 + 1
            a0 = jnp.zeros((LANES,), jnp.float32)
            a1 = jnp.zeros((LANES,), jnp.float32)
            for idx, cx, cy in ((i00, 0, 0), (i01, 0, 1), (i10, 1, 0), (i11, 1, 1)):
                v = plsc.load_gather(st_v, [idx])
                a0, a1 = _unpack_fma(v, wxs[cx] * wys[cy], a0, a1)
            acc_v[2 + 2 * l, pl.ds(s, LANES)] = a0
            acc_v[3 + 2 * l, pl.ds(s, LANES)] = a1
        return c2

    lax.fori_loop(0, CHUNK // LANES, grp_s, 0)


def _phase_c(cbase, w_v, rpk_v, acc_v, enc_hbm):
    def grp_c(g, c2):
        s = 16 * g
        for l in range(N_STAGED, N_LEVELS):
            a0 = jnp.zeros((LANES,), jnp.float32)
            a1 = jnp.zeros((LANES,), jnp.float32)
            for c in range(4):
                j = 4 * (l - N_STAGED) + c
                wv = w_v[pl.ds(j * CHUNK + s, LANES)]
                v = rpk_v[pl.ds(j * CHUNK + s, LANES)]
                a0, a1 = _unpack_fma(v, wv, a0, a1)
            acc_v[2 + 2 * l, pl.ds(s, LANES)] = a0
            acc_v[3 + 2 * l, pl.ds(s, LANES)] = a1
        return c2

    lax.fori_loop(0, CHUNK // LANES, grp_c, 0)
    pltpu.sync_copy(acc_v, enc_hbm.at[:, pl.ds(cbase, CHUNK)])


def _enc_body(nchunk, xs_hbm, ys_hbm, tpk_hbm, st_hbm, enc_hbm,
              xv0, yv0, idx0, w0_, rpk0, xv1, yv1, idx1, w1_, rpk1,
              st_v, acc_v, sem0, sem1):
    pts_per_w = nchunk * CHUNK
    wid = lax.axis_index("s") * NC + lax.axis_index("c")
    base = wid * pts_per_w

    pltpu.sync_copy(st_hbm, st_v)

    def load_x(ci, xv_v, yv_v):
        cbase = base + ci * CHUNK
        pltpu.sync_copy(xs_hbm.at[pl.ds(cbase, CHUNK)], xv_v)
        pltpu.sync_copy(ys_hbm.at[pl.ds(cbase, CHUNK)], yv_v)

    def fire(idx_v, rpk_v, sem):
        pltpu.async_copy(tpk_hbm.at[idx_v], rpk_v, sem)

    def drain(rpk_v, sem):
        pltpu.make_async_copy(tpk_hbm.at[pl.ds(0, NJH * CHUNK)], rpk_v, sem).wait()

    load_x(0, xv0, yv0)
    _phase_a(xv0, yv0, idx0, w0_)
    fire(idx0, rpk0, sem0)

    def body(ci2, carry):
        ci = 2 * ci2
        load_x(ci + 1, xv1, yv1)
        _phase_a(xv1, yv1, idx1, w1_)
        fire(idx1, rpk1, sem1)
        _phase_s(xv0, yv0, st_v, acc_v)
        drain(rpk0, sem0)
        _phase_c(base + ci * CHUNK, w0_, rpk0, acc_v, enc_hbm)

        @pl.when(ci2 + 1 < nchunk // 2)
        def _():
            load_x(ci + 2, xv0, yv0)
            _phase_a(xv0, yv0, idx0, w0_)
            fire(idx0, rpk0, sem0)

        _phase_s(xv1, yv1, st_v, acc_v)
        drain(rpk1, sem1)
        _phase_c(base + (ci + 1) * CHUNK, w1_, rpk1, acc_v, enc_hbm)
        return carry

    lax.fori_loop(0, nchunk // 2, body, 0)


def _encode(xs, ys, tpk, st, n):
    nchunk = n // (NW * CHUNK)
    assert nchunk % 2 == 0
    mesh = plsc.VectorSubcoreMesh(core_axis_name="c", subcore_axis_name="s")
    return pl.kernel(
        functools.partial(_enc_body, nchunk),
        out_type=jax.ShapeDtypeStruct((IN_DIM, n), jnp.float32),
        mesh=mesh,
        compiler_params=pltpu.CompilerParams(needs_layout_passes=False),
        scratch_types=[
            pltpu.VMEM((CHUNK,), jnp.float32), pltpu.VMEM((CHUNK,), jnp.float32),
            pltpu.VMEM((NJH * CHUNK,), jnp.int32),
            pltpu.VMEM((NJH * CHUNK,), jnp.float32),
            pltpu.VMEM((NJH * CHUNK,), jnp.int32),
            pltpu.VMEM((CHUNK,), jnp.float32), pltpu.VMEM((CHUNK,), jnp.float32),
            pltpu.VMEM((NJH * CHUNK,), jnp.int32),
            pltpu.VMEM((NJH * CHUNK,), jnp.float32),
            pltpu.VMEM((NJH * CHUNK,), jnp.int32),
            pltpu.VMEM((N_ST,), jnp.int32),
            pltpu.VMEM((IN_DIM, CHUNK), jnp.float32),
            pltpu.SemaphoreType.DMA,
            pltpu.SemaphoreType.DMA,
        ],
    )(xs, ys, tpk, st)


def _mlp_body(enc_ref, w0_ref, b0_ref, w1_ref, b1_ref, w2_ref, b2_ref, out_ref):
    prec = lax.Precision.DEFAULT
    encT = enc_ref[...]
    h = lax.dot_general(encT, w0_ref[...], (((0,), (0,)), ((), ())),
                        preferred_element_type=jnp.float32, precision=prec)
    h = jnp.maximum(h + b0_ref[...], 0.0)
    h = lax.dot_general(h, w1_ref[...], (((1,), (0,)), ((), ())),
                        preferred_element_type=jnp.float32, precision=prec)
    h = jnp.maximum(h + b1_ref[...], 0.0)
    out = lax.dot_general(h, w2_ref[...], (((1,), (0,)), ((), ())),
                          preferred_element_type=jnp.float32, precision=prec)
    out_ref[...] = out + b2_ref[...]


def _mlp(enc, W0, b0, W1, b1, W2, b2, n):
    bn = 8192
    return pl.pallas_call(
        _mlp_body,
        grid=(n // bn,),
        in_specs=[
            pl.BlockSpec((IN_DIM, bn), lambda i: (0, i)),
            pl.BlockSpec((IN_DIM, N_NEURONS), lambda i: (0, 0)),
            pl.BlockSpec((1, N_NEURONS), lambda i: (0, 0)),
            pl.BlockSpec((N_NEURONS, N_NEURONS), lambda i: (0, 0)),
            pl.BlockSpec((1, N_NEURONS), lambda i: (0, 0)),
            pl.BlockSpec((N_NEURONS, 3), lambda i: (0, 0)),
            pl.BlockSpec((1, 3), lambda i: (0, 0)),
        ],
        out_specs=pl.BlockSpec((bn, 3), lambda i: (i, 0)),
        out_shape=jax.ShapeDtypeStruct((n, 3), jnp.float32),
    )(enc, W0, b0, W1, b1, W2, b2)


def kernel(x, table, W0, b0, W1, b1, W2, b2):
    n = x.shape[0]
    xs = x[:, 0]
    ys = x[:, 1]
    tpk = lax.bitcast_convert_type(
        table.astype(jnp.bfloat16).reshape(N_LEVELS * T, F_PER_LEVEL), jnp.int32)
    st = jnp.concatenate(
        [tpk[l * T:l * T + STAGED_ROWS[l]] for l in range(N_STAGED)])
    enc = _encode(xs, ys, tpk, st, n)
    return _mlp(enc, W0, b0.reshape(1, -1), W1, b1.reshape(1, -1),
                W2, b2.reshape(1, -1), n)


# speedup vs baseline: 1.7256x; 1.0575x over previous
import functools

import numpy as np
import jax
import jax.numpy as jnp
from jax import lax
from jax.experimental import pallas as pl
from jax.experimental.pallas import tpu as pltpu
from jax.experimental.pallas import tpu_sc as plsc

N_LEVELS = 16
F_PER_LEVEL = 2
LOG2_T = 19
T = 1 << LOG2_T
BASE_RES = 16
PER_LEVEL_SCALE = 1.5
N_NEURONS = 64
IN_DIM = 2 + N_LEVELS * F_PER_LEVEL

RES = [int(np.floor(BASE_RES * (PER_LEVEL_SCALE ** l))) for l in range(N_LEVELS)]
HASHED = [(r + 1) * (r + 1) > T for r in RES]
PRIME1 = int(np.uint32(2654435761).view(np.int32))

NC, NS, LANES = 2, 16, 16
NW = NC * NS
NJ = N_LEVELS * 4
CHUNK = 128

N_STAGED = 7
STAGED_ROWS = [(RES[l] + 1) * (RES[l] + 1) for l in range(N_STAGED)]
SBASE = np.cumsum([0] + STAGED_ROWS).tolist()
N_ST = SBASE[-1]
NJS = N_STAGED * 4
NJH = NJ - NJS


def _grid_setup(xv, yv, l):
    res = RES[l]
    px = xv * float(res)
    py = yv * float(res)
    ix = px.astype(jnp.int32)
    iy = py.astype(jnp.int32)
    fx = px - ix.astype(jnp.float32)
    fy = py - iy.astype(jnp.float32)
    return ix, iy, (1.0 - fx, fx), (1.0 - fy, fy)


_HIMASK = -65536


def _unpack_fma(v, wv, a0, a1):
    f0 = plsc.bitcast(v << 16, jnp.float32)
    f1 = plsc.bitcast(v & jnp.int32(_HIMASK), jnp.float32)
    return a0 + wv * f0, a1 + wv * f1


def _phase_a(xv_v, yv_v, idx_v, w_v):
    def grp_a(g, c2):
        s = 16 * g
        xv = xv_v[pl.ds(s, LANES)]
        yv = yv_v[pl.ds(s, LANES)]
        for l in range(N_STAGED, N_LEVELS):
            ix, iy, wxs, wys = _grid_setup(xv, yv, l)
            off = l * T
            if HASHED[l]:
                hy0 = iy * PRIME1
                hy1 = (iy + 1) * PRIME1
                i00 = ((ix ^ hy0) & (T - 1)) + off
                i01 = ((ix ^ hy1) & (T - 1)) + off
                ix1 = ix + 1
                i10 = ((ix1 ^ hy0) & (T - 1)) + off
                i11 = ((ix1 ^ hy1) & (T - 1)) + off
            else:
                stride = RES[l] + 1
                i00 = ix + iy * stride + off
                i10 = i00 + 1
                i01 = i00 + stride
                i11 = i01 + 1
            corners = ((i00, 0, 0), (i01, 0, 1), (i10, 1, 0), (i11, 1, 1))
            for c, (idx, cx, cy) in enumerate(corners):
                j = 4 * (l - N_STAGED) + c
                idx_v[pl.ds(j * CHUNK + s, LANES)] = idx
                w_v[pl.ds(j * CHUNK + s, LANES)] = wxs[cx] * wys[cy]
        return c2

    lax.fori_loop(0, CHUNK // LANES, grp_a, 0)


def _phase_s(xv_v, yv_v, st_v, acc_v):
    def grp_s(g, c2):
        s = 16 * g
        xv = xv_v[pl.ds(s, LANES)]
        yv = yv_v[pl.ds(s, LANES)]
        acc_v[0, pl.ds(s, LANES)] = xv
        acc_v[1, pl.ds(s, LANES)] = yv
        for l in range(N_STAGED):
            ix, iy, wxs, wys = _grid_setup(xv, yv, l)
            stride = RES[l] + 1
            i00 = ix + iy * stride + SBASE[l]
            i10 = i00 + 1
            i01 = i00 + stride
            i11 = i01 + 1
            a0 = jnp.zeros((LANES,), jnp.float32)
            a1 = jnp.zeros((LANES,), jnp.float32)
            for idx, cx, cy in ((i00, 0, 0), (i01, 0, 1), (i10, 1, 0), (i11, 1, 1)):
                v = plsc.load_gather(st_v, [idx])
                a0, a1 = _unpack_fma(v, wxs[cx] * wys[cy], a0, a1)
            acc_v[2 + 2 * l, pl.ds(s, LANES)] = a0
            acc_v[3 + 2 * l, pl.ds(s, LANES)] = a1
        return c2

    lax.fori_loop(0, CHUNK // LANES, grp_s, 0)


def _phase_c(cbase, w_v, rpk_v, acc_v, enc_hbm):
    def grp_c(g, c2):
        s = 16 * g
        for l in range(N_STAGED, N_LEVELS):
            a0 = jnp.zeros((LANES,), jnp.float32)
            a1 = jnp.zeros((LANES,), jnp.float32)
            for c in range(4):
                j = 4 * (l - N_STAGED) + c
                wv = w_v[pl.ds(j * CHUNK + s, LANES)]
                v = rpk_v[pl.ds(j * CHUNK + s, LANES)]
                a0, a1 = _unpack_fma(v, wv, a0, a1)
            acc_v[2 + 2 * l, pl.ds(s, LANES)] = a0
            acc_v[3 + 2 * l, pl.ds(s, LANES)] = a1
        return c2

    lax.fori_loop(0, CHUNK // LANES, grp_c, 0)
    pltpu.sync_copy(acc_v, enc_hbm.at[:, pl.ds(cbase, CHUNK)])


def _enc_body(nchunk, xs_hbm, ys_hbm, tpk_hbm, st_hbm, enc_hbm,
              xv0, yv0, idx0, w0_, rpk0, xv1, yv1, idx1, w1_, rpk1,
              st_v, acc_v, sem0, sem1):
    pts_per_w = nchunk * CHUNK
    wid = lax.axis_index("s") * NC + lax.axis_index("c")
    base = wid * pts_per_w

    pltpu.sync_copy(st_hbm, st_v)

    def load_x(ci, xv_v, yv_v):
        cbase = base + ci * CHUNK
        pltpu.sync_copy(xs_hbm.at[pl.ds(cbase, CHUNK)], xv_v)
        pltpu.sync_copy(ys_hbm.at[pl.ds(cbase, CHUNK)], yv_v)

    def fire(idx_v, rpk_v, sem):
        pltpu.async_copy(tpk_hbm.at[idx_v], rpk_v, sem)

    def drain(rpk_v, sem):
        pltpu.make_async_copy(tpk_hbm.at[pl.ds(0, NJH * CHUNK)], rpk_v, sem).wait()

    load_x(0, xv0, yv0)
    _phase_a(xv0, yv0, idx0, w0_)
    fire(idx0, rpk0, sem0)

    def body(ci2, carry):
        ci = 2 * ci2
        load_x(ci + 1, xv1, yv1)
        _phase_a(xv1, yv1, idx1, w1_)
        fire(idx1, rpk1, sem1)
        _phase_s(xv0, yv0, st_v, acc_v)
        drain(rpk0, sem0)
        _phase_c(base + ci * CHUNK, w0_, rpk0, acc_v, enc_hbm)

        @pl.when(ci2 + 1 < nchunk // 2)
        def _():
            load_x(ci + 2, xv0, yv0)
            _phase_a(xv0, yv0, idx0, w0_)
            fire(idx0, rpk0, sem0)

        _phase_s(xv1, yv1, st_v, acc_v)
        drain(rpk1, sem1)
        _phase_c(base + (ci + 1) * CHUNK, w1_, rpk1, acc_v, enc_hbm)
        return carry

    lax.fori_loop(0, nchunk // 2, body, 0)


def _encode(xs, ys, tpk, st, n):
    nchunk = n // (NW * CHUNK)
    assert nchunk % 2 == 0
    mesh = plsc.VectorSubcoreMesh(core_axis_name="c", subcore_axis_name="s")
    return pl.kernel(
        functools.partial(_enc_body, nchunk),
        out_type=jax.ShapeDtypeStruct((IN_DIM, n), jnp.float32),
        mesh=mesh,
        compiler_params=pltpu.CompilerParams(needs_layout_passes=False),
        scratch_types=[
            pltpu.VMEM((CHUNK,), jnp.float32), pltpu.VMEM((CHUNK,), jnp.float32),
            pltpu.VMEM((NJH * CHUNK,), jnp.int32),
            pltpu.VMEM((NJH * CHUNK,), jnp.float32),
            pltpu.VMEM((NJH * CHUNK,), jnp.int32),
            pltpu.VMEM((CHUNK,), jnp.float32), pltpu.VMEM((CHUNK,), jnp.float32),
            pltpu.VMEM((NJH * CHUNK,), jnp.int32),
            pltpu.VMEM((NJH * CHUNK,), jnp.float32),
            pltpu.VMEM((NJH * CHUNK,), jnp.int32),
            pltpu.VMEM((N_ST,), jnp.int32),
            pltpu.VMEM((IN_DIM, CHUNK), jnp.float32),
            pltpu.SemaphoreType.DMA,
            pltpu.SemaphoreType.DMA,
        ],
    )(xs, ys, tpk, st)


def _mlp_body(enc_ref, w0_ref, b0_ref, w1_ref, b1_ref, w2_ref, b2_ref, out_ref):
    prec = lax.Precision.DEFAULT
    encT = enc_ref[...]
    h = lax.dot_general(encT, w0_ref[...], (((0,), (0,)), ((), ())),
                        preferred_element_type=jnp.float32, precision=prec)
    h = jnp.maximum(h + b0_ref[...], 0.0)
    h = lax.dot_general(h, w1_ref[...], (((1,), (0,)), ((), ())),
                        preferred_element_type=jnp.float32, precision=prec)
    h = jnp.maximum(h + b1_ref[...], 0.0)
    out = lax.dot_general(h, w2_ref[...], (((1,), (0,)), ((), ())),
                          preferred_element_type=jnp.float32, precision=prec)
    out_ref[...] = out + b2_ref[...]


def _mlp(enc, W0, b0, W1, b1, W2, b2, n):
    bn = 8192
    return pl.pallas_call(
        _mlp_body,
        grid=(n // bn,),
        in_specs=[
            pl.BlockSpec((IN_DIM, bn), lambda i: (0, i)),
            pl.BlockSpec((IN_DIM, N_NEURONS), lambda i: (0, 0)),
            pl.BlockSpec((1, N_NEURONS), lambda i: (0, 0)),
            pl.BlockSpec((N_NEURONS, N_NEURONS), lambda i: (0, 0)),
            pl.BlockSpec((1, N_NEURONS), lambda i: (0, 0)),
            pl.BlockSpec((N_NEURONS, 3), lambda i: (0, 0)),
            pl.BlockSpec((1, 3), lambda i: (0, 0)),
        ],
        out_specs=pl.BlockSpec((bn, 3), lambda i: (i, 0)),
        out_shape=jax.ShapeDtypeStruct((n, 3), jnp.float32),
    )(enc, W0, b0, W1, b1, W2, b2)


def kernel(x, table, W0, b0, W1, b1, W2, b2):
    n = x.shape[0]
    xs = x[:, 0]
    ys = x[:, 1]
    ti = lax.bitcast_convert_type(table, jnp.int32)
    tpk = ((ti[:, :, 1] & jnp.int32(_HIMASK))
           | ((ti[:, :, 0] >> 16) & jnp.int32(0xFFFF))).reshape(-1)
    st = jnp.concatenate(
        [tpk[l * T:l * T + STAGED_ROWS[l]] for l in range(N_STAGED)])
    enc = _encode(xs, ys, tpk, st, n)
    return _mlp(enc, W0, b0.reshape(1, -1), W1, b1.reshape(1, -1),
                W2, b2.reshape(1, -1), n)
